# baseline (speedup 1.0000x reference)
import functools

import jax
import jax.numpy as jnp
from jax import lax
from jax.experimental import pallas as pl
from jax.experimental.pallas import tpu as pltpu
from jax.experimental.pallas import tpu_sc as plsc

D = 64
NHOP = 2
BATCH = 4096
K = 32
L1_DECAY = 1e-06

NC, NS = 2, 16
NW = NC * NS

PROTEIN_ROWS = 1000000
DRUG_ROWS = 100000
CELL_ROWS = 1000
CW = 16384
PROT_PAD = ((PROTEIN_ROWS + CW - 1) // CW) * CW
DRUG_PAD = ((DRUG_ROWS + CW - 1) // CW) * CW
CELL_PAD = ((CELL_ROWS + CW - 1) // CW) * CW
NPROT = 6 * BATCH * K
SLICES = 4
BATCH_S = BATCH // SLICES
NPROT_S = NPROT // SLICES
R_PER_W = NPROT_S // NW
CH = 128
NCH = R_PER_W // CH
IB = BATCH_S // NW


@functools.lru_cache(maxsize=1)
def _sc_gather_build():
    mesh = plsc.VectorSubcoreMesh(core_axis_name="c", subcore_axis_name="s",
                                  num_cores=NC, num_subcores=NS)

    @functools.partial(
        pl.kernel,
        out_type=(
            jax.ShapeDtypeStruct((NPROT_S, D), jnp.float32),
            jax.ShapeDtypeStruct((BATCH_S, D), jnp.float32),
            jax.ShapeDtypeStruct((BATCH_S, D), jnp.float32),
            jax.ShapeDtypeStruct((BATCH_S, D), jnp.float32),
        ),
        mesh=mesh,
        scratch_types=[
            pltpu.VMEM((NCH, CH), jnp.int32),
            pltpu.VMEM((CH, D), jnp.float32),
            pltpu.VMEM((IB,), jnp.int32),
            pltpu.VMEM((IB, D), jnp.float32),
            pltpu.SemaphoreType.DMA,
        ],
        compiler_params=pltpu.CompilerParams(use_tc_tiling_on_sc=False),
    )
    def sc_gather(pidx_hbm, cells_hbm, d1_hbm, d2_hbm,
                  prot_hbm, cell_emb_hbm, drug_emb_hbm,
                  prot_out, cell_out, d1_out, d2_out,
                  idx_v, rows_v, iidx_v, irows_v, sem):
        wid = lax.axis_index("s") * NC + lax.axis_index("c")
        base = wid * R_PER_W

        pltpu.sync_copy(pidx_hbm.at[wid], idx_v)

        def chunk(j, carry):
            pltpu.async_copy(prot_hbm.at[idx_v.at[j]], rows_v, sem).wait()
            pltpu.sync_copy(rows_v, prot_out.at[pl.ds(base + j * CH, CH)])
            return carry

        lax.fori_loop(0, NCH, chunk, 0)

        for ids_hbm, table_hbm, out_hbm in (
            (cells_hbm, cell_emb_hbm, cell_out),
            (d1_hbm, drug_emb_hbm, d1_out),
            (d2_hbm, drug_emb_hbm, d2_out),
        ):
            pltpu.sync_copy(ids_hbm.at[wid], iidx_v)
            pltpu.async_copy(table_hbm.at[iidx_v], irows_v, sem).wait()
            pltpu.sync_copy(irows_v, out_hbm.at[pl.ds(wid * IB, IB)])

    return sc_gather


def _tp_body(in_ref, out_ref):
    t = jnp.transpose(in_ref[...])
    h = t.shape[0] // 2
    out_ref[...] = jnp.concatenate([t[:h], t[h:]], axis=-1)


@functools.lru_cache(maxsize=None)
def _tp_call(nrows):
    nb = (nrows + CW - 1) // CW
    return pl.pallas_call(
        _tp_body,
        grid=(nb,),
        in_specs=[pl.BlockSpec((D, CW), lambda j: (0, j))],
        out_specs=pl.BlockSpec((CW // 2, 2 * D), lambda j: (j, 0)),
        out_shape=jax.ShapeDtypeStruct((nb * CW // 2, 2 * D), jnp.float32),
    )


def _remap(p):
    j = p // CW
    c = p - j * CW
    half = (c >= CW // 2).astype(jnp.int32)
    r = c - half * (CW // 2)
    return j * CW + 2 * r + half


BB = 128
NB = BATCH // BB
NB_S = BATCH_S // BB
K2 = K // 2


def _tc_body(prot_ref, ce_ref, d1_ref, d2_ref, aw_ref, ab_ref, cw_ref,
             score_ref, loss_ref, cur, i0s, cfs, f1s):
    i = pl.program_id(0)
    g = pl.program_id(1)
    blk = prot_ref[...].reshape(BB, K2, 2 * D)
    aw = aw_ref[...]
    ab = ab_ref[...]

    row_lt = lax.broadcasted_iota(jnp.int32, (2 * D, 2), 0) < D
    is_c0 = lax.broadcasted_iota(jnp.int32, (2 * D, 2), 1) == 0
    sel = (row_lt == is_c0).astype(jnp.float32)

    @pl.when(g == 0)
    def _():
        cur[...] = ce_ref[...]

    @pl.when(g == 2)
    def _():
        cur[...] = d1_ref[...]

    @pl.when(g == 4)
    def _():
        cur[...] = d2_ref[...]

    item = cur[...]
    item2 = jnp.concatenate([item, item], axis=-1)
    prod = blk * item2[:, None, :]
    c2 = lax.dot_general(prod.reshape(BB * K2, 2 * D), sel,
                         (((1,), (0,)), ((), ())),
                         preferred_element_type=jnp.float32)
    c = c2.reshape(BB, K2, 2)
    m = jnp.max(jnp.max(c, axis=2, keepdims=True), axis=1, keepdims=True)
    e = jnp.exp(c - m)
    z = jnp.sum(jnp.sum(e, axis=2, keepdims=True), axis=1, keepdims=True)
    w = e / z
    w0 = jnp.broadcast_to(w[:, :, 0:1], (BB, K2, D))
    w1 = jnp.broadcast_to(w[:, :, 1:2], (BB, K2, D))
    wexp = jnp.concatenate([w0, w1], axis=-1)
    s = jnp.sum(blk * wexp, axis=1)
    out = s[:, :D] + s[:, D:]

    cur[...] = out

    @pl.when(g % 2 == 0)
    def _():
        i0s[...] = out

    def agg(a, b):
        cat = jnp.concatenate([a, b], axis=-1)
        return lax.dot_general(cat, aw, (((1,), (1,)), ((), ())),
                               preferred_element_type=jnp.float32) + ab

    @pl.when(g == 1)
    def _():
        cfs[...] = agg(i0s[...], out)

    @pl.when(g == 3)
    def _():
        f1s[...] = agg(i0s[...], out)

    @pl.when(g == 5)
    def _():
        f2 = agg(i0s[...], out)
        f1 = f1s[...]
        cf = cfs[...]
        comb = lax.dot_general(jnp.concatenate([f1, f2], axis=-1),
                               cw_ref[...], (((1,), (1,)), ((), ())),
                               preferred_element_type=jnp.float32)
        score = jnp.sum(comb * cf, axis=-1) - jnp.sum(f1 * f2, axis=-1)
        score_ref[...] = score[None, None, :]

    sq = 0.5 * jnp.sum(blk * blk)

    @pl.when(jnp.logical_and(i == 0, g == 0))
    def _():
        loss_ref[...] = jnp.zeros((1, 1), jnp.float32)

    @pl.when(g == 0)
    def _():
        ce = ce_ref[...]
        d1 = d1_ref[...]
        d2 = d2_ref[...]
        loss_ref[...] += (0.5 * (jnp.sum(ce * ce) + jnp.sum(d1 * d1)
                                 + jnp.sum(d2 * d2))).reshape(1, 1)

    loss_ref[...] += sq.reshape(1, 1)


_tc_compute = pl.pallas_call(
    _tc_body,
    grid=(NB_S, 6),
    in_specs=[
        pl.BlockSpec((BB * K2, 2 * D), lambda i, g: (i * 6 + g, 0)),
        pl.BlockSpec((BB, D), lambda i, g: (i, 0)),
        pl.BlockSpec((BB, D), lambda i, g: (i, 0)),
        pl.BlockSpec((BB, D), lambda i, g: (i, 0)),
        pl.BlockSpec((D, 2 * D), lambda i, g: (0, 0)),
        pl.BlockSpec((1, D), lambda i, g: (0, 0)),
        pl.BlockSpec((D, 2 * D), lambda i, g: (0, 0)),
    ],
    out_specs=[
        pl.BlockSpec((1, 1, BB), lambda i, g: (i, 0, 0)),
        pl.BlockSpec((1, 1), lambda i, g: (0, 0)),
    ],
    out_shape=[
        jax.ShapeDtypeStruct((NB_S, 1, BB), jnp.float32),
        jax.ShapeDtypeStruct((1, 1), jnp.float32),
    ],
    scratch_shapes=[
        pltpu.VMEM((BB, D), jnp.float32),
        pltpu.VMEM((BB, D), jnp.float32),
        pltpu.VMEM((BB, D), jnp.float32),
        pltpu.VMEM((BB, D), jnp.float32),
    ],
)


def kernel(cells, drug1, drug2, cell_neighbors, drug1_neighbors,
           drug2_neighbors, protein_emb, cell_emb, drug_emb,
           agg_W, agg_b, comb_W):
    pidx = _remap(
        jnp.stack([cell_neighbors, drug1_neighbors, drug2_neighbors])
        .astype(jnp.int32)
        .reshape(3, NHOP, NB, BB, K)
        .transpose(2, 0, 1, 3, 4)
        .reshape(SLICES, NW, NCH, CH))

    prot_lin = _tp_call(PROTEIN_ROWS)(protein_emb.T).reshape(PROT_PAD, D)
    drug_lin = _tp_call(DRUG_ROWS)(drug_emb.T).reshape(DRUG_PAD, D)
    cell_lin = _tp_call(CELL_ROWS)(cell_emb.T).reshape(CELL_PAD, D)

    cells_r = _remap(cells.astype(jnp.int32)).reshape(SLICES, NW, IB)
    drug1_r = _remap(drug1.astype(jnp.int32)).reshape(SLICES, NW, IB)
    drug2_r = _remap(drug2.astype(jnp.int32)).reshape(SLICES, NW, IB)

    sc = _sc_gather_build()
    ab2 = agg_b.reshape(1, D)
    scores, losses = [], []
    for s in range(SLICES):
        prot_rows, ce, d1e, d2e = sc(
            pidx[s], cells_r[s], drug1_r[s], drug2_r[s],
            prot_lin, cell_lin, drug_lin,
        )
        prot_pairs = prot_rows.reshape(NPROT_S // 2, 2 * D)
        score2d, loss = _tc_compute(prot_pairs, ce, d1e, d2e, agg_W,
                                    ab2, comb_W)
        scores.append(score2d.reshape(BATCH_S))
        losses.append(loss[0, 0])
    score = jnp.concatenate(scores)
    emb_loss = (losses[0] + losses[1] + losses[2] + losses[3]) * (
        L1_DECAY / BATCH)
    return score, emb_loss


# speedup vs baseline: 1.2805x; 1.2805x over previous
import functools

import jax
import jax.numpy as jnp
from jax import lax
from jax.experimental import pallas as pl
from jax.experimental.pallas import tpu as pltpu
from jax.experimental.pallas import tpu_sc as plsc

D = 64
NHOP = 2
BATCH = 4096
K = 32
L1_DECAY = 1e-06

NC, NS = 2, 16
NW = NC * NS

PROTEIN_ROWS = 1000000
DRUG_ROWS = 100000
CELL_ROWS = 1000
CW = 16384
PROT_PAD = ((PROTEIN_ROWS + CW - 1) // CW) * CW
DRUG_PAD = ((DRUG_ROWS + CW - 1) // CW) * CW
CELL_PAD = ((CELL_ROWS + CW - 1) // CW) * CW
NPROT = 6 * BATCH * K
SLICES = 4
BATCH_S = BATCH // SLICES
NPROT_S = NPROT // SLICES
R_PER_W = NPROT_S // NW
CH = 128
NCH = R_PER_W // CH
IB = BATCH_S // NW


@functools.lru_cache(maxsize=1)
def _sc_gather_build():
    mesh = plsc.VectorSubcoreMesh(core_axis_name="c", subcore_axis_name="s",
                                  num_cores=NC, num_subcores=NS)

    @functools.partial(
        pl.kernel,
        out_type=(
            jax.ShapeDtypeStruct((NPROT_S, D), jnp.float32),
            jax.ShapeDtypeStruct((BATCH_S, D), jnp.float32),
            jax.ShapeDtypeStruct((BATCH_S, D), jnp.float32),
            jax.ShapeDtypeStruct((BATCH_S, D), jnp.float32),
        ),
        mesh=mesh,
        scratch_types=[
            pltpu.VMEM((NCH, CH), jnp.int32),
            pltpu.VMEM((CH, D), jnp.float32),
            pltpu.VMEM((IB,), jnp.int32),
            pltpu.VMEM((IB, D), jnp.float32),
            pltpu.SemaphoreType.DMA,
        ],
        compiler_params=pltpu.CompilerParams(use_tc_tiling_on_sc=False),
    )
    def sc_gather(pidx_hbm, cells_hbm, d1_hbm, d2_hbm,
                  prot_hbm, cell_emb_hbm, drug_emb_hbm,
                  prot_out, cell_out, d1_out, d2_out,
                  idx_v, rows_v, iidx_v, irows_v, sem):
        wid = lax.axis_index("s") * NC + lax.axis_index("c")
        base = wid * R_PER_W

        pltpu.sync_copy(pidx_hbm.at[wid], idx_v)

        def chunk(j, carry):
            pltpu.async_copy(prot_hbm.at[idx_v.at[j]], rows_v, sem).wait()
            pltpu.sync_copy(rows_v, prot_out.at[pl.ds(base + j * CH, CH)])
            return carry

        lax.fori_loop(0, NCH, chunk, 0)

        for ids_hbm, table_hbm, out_hbm in (
            (cells_hbm, cell_emb_hbm, cell_out),
            (d1_hbm, drug_emb_hbm, d1_out),
            (d2_hbm, drug_emb_hbm, d2_out),
        ):
            pltpu.sync_copy(ids_hbm.at[wid], iidx_v)
            pltpu.async_copy(table_hbm.at[iidx_v], irows_v, sem).wait()
            pltpu.sync_copy(irows_v, out_hbm.at[pl.ds(wid * IB, IB)])

    return sc_gather


def _tp_body(in_ref, out_ref):
    t = jnp.transpose(in_ref[...])
    h = t.shape[0] // 2
    out_ref[...] = jnp.concatenate([t[:h], t[h:]], axis=-1)


@functools.lru_cache(maxsize=None)
def _tp_call(nrows):
    nb = (nrows + CW - 1) // CW
    return pl.pallas_call(
        _tp_body,
        grid=(nb,),
        in_specs=[pl.BlockSpec((D, CW), lambda j: (0, j))],
        out_specs=pl.BlockSpec((CW // 2, 2 * D), lambda j: (j, 0)),
        out_shape=jax.ShapeDtypeStruct((nb * CW // 2, 2 * D), jnp.float32),
    )


def _remap(p):
    j = p // CW
    c = p - j * CW
    half = (c >= CW // 2).astype(jnp.int32)
    r = c - half * (CW // 2)
    return j * CW + 2 * r + half


BB = 64
NB = BATCH // BB
NB_S = BATCH_S // BB
K2 = K // 2


def _tc_body(prot_ref, ce_ref, d1_ref, d2_ref, aw_ref, ab_ref, cw_ref,
             score_ref, loss_ref):
    i = pl.program_id(0)
    n2 = prot_ref[...].reshape(6, BB, K2, 2 * D)
    ce = ce_ref[...]
    d1 = d1_ref[...]
    d2 = d2_ref[...]
    aw = aw_ref[...]
    ab = ab_ref[...]
    cw = cw_ref[...]

    row_lt = lax.broadcasted_iota(jnp.int32, (2 * D, 2), 0) < D
    is_c0 = lax.broadcasted_iota(jnp.int32, (2 * D, 2), 1) == 0
    sel = (row_lt == is_c0).astype(jnp.float32)

    def attn(item, g):
        blk = n2[g]
        item2 = jnp.concatenate([item, item], axis=-1)
        prod = blk * item2[:, None, :]
        c2 = lax.dot_general(prod.reshape(BB * K2, 2 * D), sel,
                             (((1,), (0,)), ((), ())),
                             preferred_element_type=jnp.float32)
        c = c2.reshape(BB, K2, 2)
        m = jnp.max(jnp.max(c, axis=2, keepdims=True), axis=1, keepdims=True)
        e = jnp.exp(c - m)
        z = jnp.sum(jnp.sum(e, axis=2, keepdims=True), axis=1, keepdims=True)
        w = e / z
        w0 = jnp.broadcast_to(w[:, :, 0:1], (BB, K2, D))
        w1 = jnp.broadcast_to(w[:, :, 1:2], (BB, K2, D))
        wexp = jnp.concatenate([w0, w1], axis=-1)
        s = jnp.sum(blk * wexp, axis=1)
        return s[:, :D] + s[:, D:]

    ci0 = attn(ce, 0)
    ci1 = attn(ci0, 1)
    x10 = attn(d1, 2)
    x11 = attn(x10, 3)
    x20 = attn(d2, 4)
    x21 = attn(x20, 5)

    def agg(a, b):
        cat = jnp.concatenate([a, b], axis=-1)
        return lax.dot_general(cat, aw, (((1,), (1,)), ((), ())),
                               preferred_element_type=jnp.float32) + ab

    cf = agg(ci0, ci1)
    f1 = agg(x10, x11)
    f2 = agg(x20, x21)
    comb = lax.dot_general(jnp.concatenate([f1, f2], axis=-1), cw,
                           (((1,), (1,)), ((), ())),
                           preferred_element_type=jnp.float32)
    score = jnp.sum(comb * cf, axis=-1) - jnp.sum(f1 * f2, axis=-1)
    score_ref[...] = score[None, None, :]

    sq = 0.5 * (jnp.sum(ce * ce) + jnp.sum(d1 * d1) + jnp.sum(d2 * d2)
                + jnp.sum(n2 * n2))

    @pl.when(i == 0)
    def _():
        loss_ref[...] = jnp.zeros((1, 1), jnp.float32)

    loss_ref[...] += sq.reshape(1, 1)


_tc_compute = pl.pallas_call(
    _tc_body,
    grid=(NB_S,),
    in_specs=[
        pl.BlockSpec((6 * BB * K2, 2 * D), lambda i: (i, 0)),
        pl.BlockSpec((BB, D), lambda i: (i, 0)),
        pl.BlockSpec((BB, D), lambda i: (i, 0)),
        pl.BlockSpec((BB, D), lambda i: (i, 0)),
        pl.BlockSpec((D, 2 * D), lambda i: (0, 0)),
        pl.BlockSpec((1, D), lambda i: (0, 0)),
        pl.BlockSpec((D, 2 * D), lambda i: (0, 0)),
    ],
    out_specs=[
        pl.BlockSpec((1, 1, BB), lambda i: (i, 0, 0)),
        pl.BlockSpec((1, 1), lambda i: (0, 0)),
    ],
    out_shape=[
        jax.ShapeDtypeStruct((NB_S, 1, BB), jnp.float32),
        jax.ShapeDtypeStruct((1, 1), jnp.float32),
    ],
)


def kernel(cells, drug1, drug2, cell_neighbors, drug1_neighbors,
           drug2_neighbors, protein_emb, cell_emb, drug_emb,
           agg_W, agg_b, comb_W):
    pidx = _remap(
        jnp.stack([cell_neighbors, drug1_neighbors, drug2_neighbors])
        .astype(jnp.int32)
        .reshape(3, NHOP, NB, BB, K)
        .transpose(2, 0, 1, 3, 4)
        .reshape(SLICES, NW, NCH, CH))

    prot_lin = _tp_call(PROTEIN_ROWS)(protein_emb.T).reshape(PROT_PAD, D)
    drug_lin = _tp_call(DRUG_ROWS)(drug_emb.T).reshape(DRUG_PAD, D)
    cell_lin = _tp_call(CELL_ROWS)(cell_emb.T).reshape(CELL_PAD, D)

    cells_r = _remap(cells.astype(jnp.int32)).reshape(SLICES, NW, IB)
    drug1_r = _remap(drug1.astype(jnp.int32)).reshape(SLICES, NW, IB)
    drug2_r = _remap(drug2.astype(jnp.int32)).reshape(SLICES, NW, IB)

    sc = _sc_gather_build()
    ab2 = agg_b.reshape(1, D)
    scores, losses = [], []
    for s in range(SLICES):
        prot_rows, ce, d1e, d2e = sc(
            pidx[s], cells_r[s], drug1_r[s], drug2_r[s],
            prot_lin, cell_lin, drug_lin,
        )
        prot_pairs = prot_rows.reshape(NPROT_S // 2, 2 * D)
        score2d, loss = _tc_compute(prot_pairs, ce, d1e, d2e, agg_W,
                                    ab2, comb_W)
        scores.append(score2d.reshape(BATCH_S))
        losses.append(loss[0, 0])
    score = jnp.concatenate(scores)
    emb_loss = (losses[0] + losses[1] + losses[2] + losses[3]) * (
        L1_DECAY / BATCH)
    return score, emb_loss
